# Initial kernel scaffold; baseline (speedup 1.0000x reference)
import jax
import jax.numpy as jnp
from jax.experimental import pallas as pl


def kernel(h, e, edge_index, W_Q, b_Q, W_K, b_K, W_V, b_V, W_pe, b_pe, W_ng1, b_ng1, ln_ng_w, ln_ng_b, W_ng2, b_ng2, W_cc, b_cc, ln_cc_w, ln_cc_b, conv1_w, conv1_b, conv2_w, conv2_b):
    raise NotImplementedError("write your pallas kernel here")


# speedup vs baseline: 11.7634x; 11.7634x over previous
import functools
import math

import jax
import jax.numpy as jnp
import numpy as np
from jax import lax
from jax.experimental import pallas as pl
from jax.experimental.pallas import tpu as pltpu
from jax.experimental.pallas import tpu_sc as plsc

H = 8
D = 16
OH = H * D

_HIGH = lax.Precision.HIGHEST


def _node_body(h_ref, wq, bq, wk, bk, wv, bv, wng1, bng1, lnw, lnb, wng2, bng2,
               qh_ref, kh_ref, vh_ref, ng_ref):
    hb = h_ref[...]
    qh_ref[...] = jnp.dot(hb, wq[...], precision=_HIGH) + bq[...]
    kh_ref[...] = jnp.dot(hb, wk[...], precision=_HIGH) + bk[...]
    vh_ref[...] = jnp.dot(hb, wv[...], precision=_HIGH) + bv[...]
    a = jnp.dot(hb, wng1[...], precision=_HIGH) + bng1[...]
    mu = jnp.mean(a, axis=-1, keepdims=True)
    var = jnp.mean((a - mu) ** 2, axis=-1, keepdims=True)
    a = (a - mu) * lax.rsqrt(var + 1e-5) * lnw[...] + lnb[...]
    ng = jnp.dot(jnp.tanh(a), wng2[...], precision=_HIGH) + bng2[...]
    ng_ref[...] = jax.nn.sigmoid(ng)


def _node_stage(h, W_Q, b_Q, W_K, b_K, W_V, b_V, W_ng1, b_ng1, lnw, lnb, W_ng2, b_ng2):
    N = h.shape[0]
    BN = 1000
    grid = (N // BN,)
    f32 = jnp.float32
    full = lambda r, c: pl.BlockSpec((r, c), lambda i: (0, 0))
    blk = lambda c: pl.BlockSpec((BN, c), lambda i: (i, 0))
    out_shapes = tuple(jax.ShapeDtypeStruct((N, OH), f32) for _ in range(4))
    return pl.pallas_call(
        _node_body,
        grid=grid,
        in_specs=[blk(128), full(128, OH), full(1, OH), full(128, OH), full(1, OH),
                  full(128, OH), full(1, OH), full(128, 64), full(1, 64),
                  full(1, 64), full(1, 64), full(64, OH), full(1, OH)],
        out_specs=tuple(blk(OH) for _ in range(4)),
        out_shape=out_shapes,
    )(h, W_Q, b_Q[None], W_K, b_K[None], W_V, b_V[None], W_ng1, b_ng1[None],
      lnw[None], lnb[None], W_ng2, b_ng2[None])


def _make_gather_kq(N, E, CH, per_w, NC, NS):
    f32 = jnp.float32
    mesh = plsc.VectorSubcoreMesh(core_axis_name="c", subcore_axis_name="s")

    @functools.partial(
        pl.kernel,
        out_type=(jax.ShapeDtypeStruct((E, OH), f32),
                  jax.ShapeDtypeStruct((E, OH), f32)),
        mesh=mesh,
        scratch_types=[
            pltpu.VMEM((CH,), jnp.int32), pltpu.VMEM((CH,), jnp.int32),
            pltpu.VMEM((CH, OH), f32), pltpu.VMEM((CH, OH), f32),
            pltpu.SemaphoreType.DMA, pltpu.SemaphoreType.DMA,
        ],
    )
    def gather_kq(kh_hbm, qh_hbm, src2d, dst2d, ksrc_hbm, qdst_hbm,
                  sidx, didx, krows, qrows, sem1, sem2):
        wid = lax.axis_index("s") * NC + lax.axis_index("c")

        def body(i, _):
            c = wid * per_w + i
            pltpu.sync_copy(src2d.at[c], sidx)
            pltpu.sync_copy(dst2d.at[c], didx)
            cp1 = pltpu.async_copy(kh_hbm.at[sidx], krows, sem1)
            cp2 = pltpu.async_copy(qh_hbm.at[didx], qrows, sem2)
            cp1.wait()
            cp2.wait()
            pltpu.sync_copy(krows, ksrc_hbm.at[pl.ds(c * CH, CH)])
            pltpu.sync_copy(qrows, qdst_hbm.at[pl.ds(c * CH, CH)])
            return _

        lax.fori_loop(0, per_w, body, None)

    return gather_kq


def _edge_body(e_ref, k_ref, q_ref, wpe, bpe, bd1, bd2, bd3, bcc, gm, lnw, lnb,
               m12, b12, summ, eout_ref, gate_ref):
    K = k_ref[...]
    Q = q_ref[...]
    P = jnp.dot(e_ref[...], wpe[...], precision=_HIGH) + bpe[...]
    score = K * Q * P * 0.25
    eout_ref[...] = score
    cc = (jnp.dot(K, bd1[...], precision=_HIGH)
          + jnp.dot(Q, bd2[...], precision=_HIGH)
          + jnp.dot(P, bd3[...], precision=_HIGH) + bcc[...])
    g = gm[...]
    mu = jnp.dot(cc, g, precision=_HIGH)
    var = jnp.dot(cc * cc, g, precision=_HIGH) - mu * mu
    t = jnp.tanh((cc - mu) * lax.rsqrt(var + 1e-5) * lnw[...] + lnb[...])
    u = jnp.dot(t, m12[...], precision=_HIGH) + b12[...]
    gs = score * jax.nn.sigmoid(u)
    s = jnp.dot(gs, summ[...], precision=_HIGH)
    s = jnp.clip(s, -0.005, 0.005)
    col = lax.broadcasted_iota(jnp.int32, s.shape, 1)
    gate_ref[...] = jnp.where(col < H, jnp.exp(s), 0.0)


def _edge_stage(e, ksrc, qdst, W_pe, b_pe, bd1, bd2, bd3, bcc, gm, lnw, lnb,
                m12, b12, summ):
    E = e.shape[0]
    BE = 512
    grid = (E // BE,)
    f32 = jnp.float32
    full = lambda r, c: pl.BlockSpec((r, c), lambda i: (0, 0))
    blk = lambda c: pl.BlockSpec((BE, c), lambda i: (i, 0))
    return pl.pallas_call(
        _edge_body,
        grid=grid,
        in_specs=[blk(128), blk(OH), blk(OH), full(128, OH), full(1, OH),
                  full(OH, OH), full(OH, OH), full(OH, OH), full(1, OH),
                  full(OH, OH), full(1, OH), full(1, OH), full(OH, OH),
                  full(1, OH), full(OH, D)],
        out_specs=(blk(OH), blk(D)),
        out_shape=(jax.ShapeDtypeStruct((E, OH), f32),
                   jax.ShapeDtypeStruct((E, D), f32)),
    )(e, ksrc, qdst, W_pe, b_pe[None], bd1, bd2, bd3, bcc, gm, lnw, lnb, m12,
      b12, summ)


def _make_scatter(NPAD, E, CH, per_w, NC, NS):
    f32 = jnp.float32
    mesh = plsc.VectorSubcoreMesh(core_axis_name="c", subcore_axis_name="s")
    rows_per_tile = NPAD // NS

    @functools.partial(
        pl.kernel,
        out_type=jax.ShapeDtypeStruct((NC * NPAD, OH), f32),
        mesh=mesh,
        scratch_types=[
            pltpu.VMEM((CH,), jnp.int32), pltpu.VMEM((CH,), jnp.int32),
            pltpu.VMEM((CH, OH), f32), pltpu.VMEM((CH * D,), f32),
            pltpu.VMEM_SHARED((NPAD, OH), f32),
            pltpu.SemaphoreType.DMA,
        ],
    )
    def scatter_wv(vh_hbm, src2d, dst2d, gatesf_hbm, zw_hbm,
                   wv_out, sidx, didx, vrows, grows, wv_sh, sem):
        cid = lax.axis_index("c")
        sid = lax.axis_index("s")
        wid = sid * NC + cid
        r0 = sid * rows_per_tile
        nz = rows_per_tile // CH
        pltpu.sync_copy(zw_hbm, vrows)

        def zbody(j, _):
            pltpu.sync_copy(vrows, wv_sh.at[pl.ds(r0 + j * CH, CH)])
            return _

        lax.fori_loop(0, nz, zbody, None)
        plsc.subcore_barrier()

        def body(i, _):
            c = wid * per_w + i
            pltpu.sync_copy(src2d.at[c], sidx)
            pltpu.sync_copy(dst2d.at[c], didx)
            cp = pltpu.async_copy(vh_hbm.at[sidx], vrows, sem)
            pltpu.sync_copy(gatesf_hbm.at[pl.ds(c * CH * D, CH * D)], grows)
            cp.wait()

            def edge_mul(ei, _):
                g16 = grows[pl.ds(ei * D, D)]
                for hh in range(H):
                    sl = pl.ds(hh * D, D)
                    vrows[ei, sl] = vrows[ei, sl] * g16[hh]
                return _

            lax.fori_loop(0, CH, edge_mul, None)
            pltpu.sync_copy(vrows, wv_sh.at[didx], add=True)
            return _

        lax.fori_loop(0, per_w, body, None)
        plsc.subcore_barrier()

        def obody(j, _):
            pltpu.sync_copy(wv_sh.at[pl.ds(r0 + j * CH, CH)], vrows)
            pltpu.sync_copy(vrows, wv_out.at[pl.ds(cid * NPAD + r0 + j * CH, CH)])
            return _

        lax.fori_loop(0, nz, obody, None)

    return scatter_wv


def _make_zsum(NPAD, E, CH, per_w, NC, NS):
    f32 = jnp.float32
    mesh = plsc.VectorSubcoreMesh(core_axis_name="c", subcore_axis_name="s")
    NW = NC * NS
    ZLEN = NPAD * H

    @functools.partial(
        pl.kernel,
        out_type=jax.ShapeDtypeStruct((NW * ZLEN,), f32),
        mesh=mesh,
        scratch_types=[
            pltpu.VMEM((CH,), jnp.int32), pltpu.VMEM((CH * D,), f32),
            pltpu.VMEM((ZLEN,), f32),
        ],
        compiler_params=pltpu.CompilerParams(needs_layout_passes=False),
    )
    def zsum(dst2d, gatesf_hbm, zzf_hbm, z_out, didx, grows, zpart):
        cid = lax.axis_index("c")
        sid = lax.axis_index("s")
        wid = sid * NC + cid
        lane = lax.iota(jnp.int32, 16)
        pltpu.sync_copy(zzf_hbm, zpart)

        def body(i, _):
            c = wid * per_w + i
            pltpu.sync_copy(dst2d.at[c], didx)
            pltpu.sync_copy(gatesf_hbm.at[pl.ds(c * CH * D, CH * D)], grows)

            def zqbody(q, _):
                dvec = didx[pl.ds(q * 16, 16)]
                for r in range(16):
                    dsel = jnp.take_along_axis(
                        dvec, jnp.broadcast_to(jnp.int32(r), (16,)), axis=0)
                    zidx = dsel * H + lane
                    gvals = grows[pl.ds((q * 16 + r) * D, 16)]
                    cur = plsc.load_gather(zpart, [zidx])
                    plsc.store_scatter(zpart, [zidx], cur + gvals)
                return _

            lax.fori_loop(0, CH // 16, zqbody, None)
            return _

        lax.fori_loop(0, per_w, body, None)
        pltpu.sync_copy(zpart, z_out.at[pl.ds(wid * ZLEN, ZLEN)])

    return zsum


def _combine_body(wv0, wv1, z3, ng, expm, hout_ref):
    wv = (wv0[...] + wv1[...]) * ng[...]
    z8 = jnp.sum(z3[...], axis=0)
    z = jnp.dot(z8, expm[...], precision=_HIGH)
    hout_ref[...] = wv / (z + 1e-30)


def _combine_stage(wv0, wv1, z3, ngate, expm):
    N = wv0.shape[0]
    NW = z3.shape[0]
    BN = 1024
    grid = (N // BN,)
    full = lambda r, c: pl.BlockSpec((r, c), lambda i: (0, 0))
    blk = lambda c: pl.BlockSpec((BN, c), lambda i: (i, 0))
    return pl.pallas_call(
        _combine_body,
        grid=grid,
        in_specs=[blk(OH), blk(OH),
                  pl.BlockSpec((NW, BN, H), lambda i: (0, i, 0)),
                  blk(OH), full(H, OH)],
        out_specs=blk(OH),
        out_shape=jax.ShapeDtypeStruct((N, OH), jnp.float32),
    )(wv0, wv1, z3, ngate, expm)


def kernel(h, e, edge_index, W_Q, b_Q, W_K, b_K, W_V, b_V, W_pe, b_pe,
           W_ng1, b_ng1, ln_ng_w, ln_ng_b, W_ng2, b_ng2, W_cc, b_cc,
           ln_cc_w, ln_cc_b, conv1_w, conv1_b, conv2_w, conv2_b):
    f32 = jnp.float32
    N = h.shape[0]
    E = e.shape[0]

    I8 = jnp.eye(H, dtype=f32)
    bd1 = jnp.kron(I8, W_cc[0:16, :])
    bd2 = jnp.kron(I8, W_cc[16:32, :])
    bd3 = jnp.kron(I8, W_cc[32:48, :])
    gm = jnp.kron(I8, jnp.full((D, D), 1.0 / D, dtype=f32))
    bcc = jnp.tile(b_cc, H)[None]
    lnw = jnp.tile(ln_cc_w, H)[None]
    lnb = jnp.tile(ln_cc_b, H)[None]
    eyes = [jnp.eye(D, k=1 - k, dtype=f32) for k in range(3)]
    M1 = sum(jnp.kron(conv1_w[:, :, k].T, eyes[k]) for k in range(3))
    M2 = sum(jnp.kron(conv2_w[:, :, k].T, eyes[k]) for k in range(3))
    b1v = jnp.repeat(conv1_b, D)
    b2v = jnp.repeat(conv2_b, D)
    m12 = M1 @ M2
    b12 = (b1v @ M2 + b2v)[None]
    summ = jnp.concatenate(
        [jnp.kron(I8, jnp.ones((D, 1), f32)), jnp.zeros((OH, H), f32)], axis=1)
    expm = jnp.kron(I8, jnp.ones((1, D), f32))

    info = plsc.get_sparse_core_info()
    NC, NS = info.num_cores, info.num_subcores
    NW = NC * NS
    CH = 80
    nchunk = E // CH
    per_w = nchunk // NW
    src2d = edge_index[0].reshape(nchunk, CH)
    dst2d = edge_index[1].reshape(nchunk, CH)

    qh, kh, vh, ngate = _node_stage(h, W_Q, b_Q, W_K, b_K, W_V, b_V,
                                    W_ng1, b_ng1, ln_ng_w, ln_ng_b, W_ng2, b_ng2)
    ksrc, qdst = _make_gather_kq(N, E, CH, per_w, NC, NS)(kh, qh, src2d, dst2d)
    e_out, gates = _edge_stage(e, ksrc, qdst, W_pe, b_pe, bd1, bd2, bd3, bcc,
                               gm, lnw, lnb, m12, b12, summ)
    NPAD = ((N + 8 * NW - 1) // (8 * NW)) * (8 * NW)
    zw = jnp.zeros((CH, OH), f32)
    zzf = jnp.zeros((NPAD * H,), f32)
    gatesf = gates.reshape(E * D)
    wv_p = _make_scatter(NPAD, E, CH, per_w, NC, NS)(
        vh, src2d, dst2d, gatesf, zw)
    z_p = _make_zsum(NPAD, E, CH, per_w, NC, NS)(dst2d, gatesf, zzf)
    ng_pad = jnp.concatenate([ngate, jnp.zeros((NPAD - N, OH), f32)], axis=0)
    z3 = z_p.reshape(NW, NPAD, H)
    h_out = _combine_stage(wv_p[:NPAD], wv_p[NPAD:], z3, ng_pad, expm)

    return (h_out[:N].reshape(N, H, D), e_out.reshape(E, H, D))


# speedup vs baseline: 18.5629x; 1.5780x over previous
import functools
import math

import jax
import jax.numpy as jnp
import numpy as np
from jax import lax
from jax.experimental import pallas as pl
from jax.experimental.pallas import tpu as pltpu
from jax.experimental.pallas import tpu_sc as plsc

H = 8
D = 16
OH = H * D

_HIGH = lax.Precision.HIGHEST


def _node_body(h_ref, wq, bq, wk, bk, wv, bv, wng1, bng1, lnw, lnb, wng2, bng2,
               qh_ref, kh_ref, vh_ref, ng_ref):
    hb = h_ref[...]
    qh_ref[...] = jnp.dot(hb, wq[...], precision=_HIGH) + bq[...]
    kh_ref[...] = jnp.dot(hb, wk[...], precision=_HIGH) + bk[...]
    vh_ref[...] = jnp.dot(hb, wv[...], precision=_HIGH) + bv[...]
    a = jnp.dot(hb, wng1[...], precision=_HIGH) + bng1[...]
    mu = jnp.mean(a, axis=-1, keepdims=True)
    var = jnp.mean((a - mu) ** 2, axis=-1, keepdims=True)
    a = (a - mu) * lax.rsqrt(var + 1e-5) * lnw[...] + lnb[...]
    ng = jnp.dot(jnp.tanh(a), wng2[...], precision=_HIGH) + bng2[...]
    ng_ref[...] = jax.nn.sigmoid(ng)


def _node_stage(h, W_Q, b_Q, W_K, b_K, W_V, b_V, W_ng1, b_ng1, lnw, lnb, W_ng2, b_ng2):
    N = h.shape[0]
    BN = 1000
    grid = (N // BN,)
    f32 = jnp.float32
    full = lambda r, c: pl.BlockSpec((r, c), lambda i: (0, 0))
    blk = lambda c: pl.BlockSpec((BN, c), lambda i: (i, 0))
    out_shapes = tuple(jax.ShapeDtypeStruct((N, OH), f32) for _ in range(4))
    return pl.pallas_call(
        _node_body,
        grid=grid,
        in_specs=[blk(128), full(128, OH), full(1, OH), full(128, OH), full(1, OH),
                  full(128, OH), full(1, OH), full(128, 64), full(1, 64),
                  full(1, 64), full(1, 64), full(64, OH), full(1, OH)],
        out_specs=tuple(blk(OH) for _ in range(4)),
        out_shape=out_shapes,
    )(h, W_Q, b_Q[None], W_K, b_K[None], W_V, b_V[None], W_ng1, b_ng1[None],
      lnw[None], lnb[None], W_ng2, b_ng2[None])


def _make_gather_kq(N, E, CH, per_w, NC, NS):
    f32 = jnp.float32
    mesh = plsc.VectorSubcoreMesh(core_axis_name="c", subcore_axis_name="s")

    @functools.partial(
        pl.kernel,
        out_type=(jax.ShapeDtypeStruct((E, OH), f32),
                  jax.ShapeDtypeStruct((E, OH), f32)),
        mesh=mesh,
        scratch_types=[
            pltpu.VMEM((CH,), jnp.int32), pltpu.VMEM((CH,), jnp.int32),
            pltpu.VMEM((CH, OH), f32), pltpu.VMEM((CH, OH), f32),
            pltpu.SemaphoreType.DMA, pltpu.SemaphoreType.DMA,
        ],
    )
    def gather_kq(kh_hbm, qh_hbm, src2d, dst2d, ksrc_hbm, qdst_hbm,
                  sidx, didx, krows, qrows, sem1, sem2):
        wid = lax.axis_index("s") * NC + lax.axis_index("c")

        def body(i, _):
            c = wid * per_w + i
            pltpu.sync_copy(src2d.at[c], sidx)
            pltpu.sync_copy(dst2d.at[c], didx)
            cp1 = pltpu.async_copy(kh_hbm.at[sidx], krows, sem1)
            cp2 = pltpu.async_copy(qh_hbm.at[didx], qrows, sem2)
            cp1.wait()
            cp2.wait()
            pltpu.sync_copy(krows, ksrc_hbm.at[pl.ds(c * CH, CH)])
            pltpu.sync_copy(qrows, qdst_hbm.at[pl.ds(c * CH, CH)])
            return _

        lax.fori_loop(0, per_w, body, None)

    return gather_kq


def _edge_body(e_ref, k_ref, q_ref, wpe, bpe, bd1, bd2, bd3, bcc, gm, lnw, lnb,
               m12, b12, summ, eout_ref, gate_ref):
    bf16 = jnp.bfloat16
    f32 = jnp.float32
    K = k_ref[...]
    Q = q_ref[...]
    P = jnp.dot(e_ref[...], wpe[...], precision=_HIGH) + bpe[...]
    score = K * Q * P * 0.25
    eout_ref[...] = score
    cc = (jnp.dot(K.astype(bf16), bd1[...], preferred_element_type=f32)
          + jnp.dot(Q.astype(bf16), bd2[...], preferred_element_type=f32)
          + jnp.dot(P.astype(bf16), bd3[...], preferred_element_type=f32)
          + bcc[...])
    g = gm[...]
    mu = jnp.dot(cc.astype(bf16), g, preferred_element_type=f32)
    var = jnp.dot((cc * cc).astype(bf16), g, preferred_element_type=f32) - mu * mu
    t = jnp.tanh((cc - mu) * lax.rsqrt(var + 1e-5) * lnw[...] + lnb[...])
    u = jnp.dot(t.astype(bf16), m12[...], preferred_element_type=f32) + b12[...]
    gs = score * jax.nn.sigmoid(u)
    s = jnp.dot(gs.astype(bf16), summ[...], preferred_element_type=f32)
    s = jnp.clip(s, -0.005, 0.005)
    col = lax.broadcasted_iota(jnp.int32, s.shape, 1)
    gate_ref[...] = jnp.where(col < H, jnp.exp(s), 0.0)


def _edge_stage(e, ksrc, qdst, W_pe, b_pe, bd1, bd2, bd3, bcc, gm, lnw, lnb,
                m12, b12, summ):
    E = e.shape[0]
    BE = 512
    grid = (E // BE,)
    f32 = jnp.float32
    full = lambda r, c: pl.BlockSpec((r, c), lambda i: (0, 0))
    blk = lambda c: pl.BlockSpec((BE, c), lambda i: (i, 0))
    return pl.pallas_call(
        _edge_body,
        grid=grid,
        in_specs=[blk(128), blk(OH), blk(OH), full(128, OH), full(1, OH),
                  full(OH, OH), full(OH, OH), full(OH, OH), full(1, OH),
                  full(OH, OH), full(1, OH), full(1, OH), full(OH, OH),
                  full(1, OH), full(OH, D)],
        out_specs=(blk(OH), blk(D)),
        out_shape=(jax.ShapeDtypeStruct((E, OH), f32),
                   jax.ShapeDtypeStruct((E, D), f32)),
    )(e, ksrc, qdst, W_pe, b_pe[None], bd1, bd2, bd3, bcc, gm, lnw, lnb, m12,
      b12, summ)


def _make_scatter(NPAD, E, CH, per_w, NC, NS):
    f32 = jnp.float32
    mesh = plsc.VectorSubcoreMesh(core_axis_name="c", subcore_axis_name="s")
    rows_per_tile = NPAD // NS

    @functools.partial(
        pl.kernel,
        out_type=jax.ShapeDtypeStruct((NC * NPAD, OH), f32),
        mesh=mesh,
        scratch_types=[
            pltpu.VMEM((CH,), jnp.int32), pltpu.VMEM((CH,), jnp.int32),
            pltpu.VMEM((CH, OH), f32), pltpu.VMEM((CH * D,), f32),
            pltpu.VMEM_SHARED((NPAD, OH), f32),
            pltpu.SemaphoreType.DMA,
        ],
    )
    def scatter_wv(vh_hbm, src2d, dst2d, gatesf_hbm, zw_hbm,
                   wv_out, sidx, didx, vrows, grows, wv_sh, sem):
        cid = lax.axis_index("c")
        sid = lax.axis_index("s")
        wid = sid * NC + cid
        r0 = sid * rows_per_tile
        nz = rows_per_tile // CH
        pltpu.sync_copy(zw_hbm, vrows)

        def zbody(j, _):
            pltpu.sync_copy(vrows, wv_sh.at[pl.ds(r0 + j * CH, CH)])
            return _

        lax.fori_loop(0, nz, zbody, None)
        plsc.subcore_barrier()

        def body(i, _):
            c = wid * per_w + i
            pltpu.sync_copy(src2d.at[c], sidx)
            pltpu.sync_copy(dst2d.at[c], didx)
            cp = pltpu.async_copy(vh_hbm.at[sidx], vrows, sem)
            pltpu.sync_copy(gatesf_hbm.at[pl.ds(c * CH * D, CH * D)], grows)
            cp.wait()

            def edge_mul(ei, _):
                g16 = grows[pl.ds(ei * D, D)]
                for hh in range(H):
                    sl = pl.ds(hh * D, D)
                    vrows[ei, sl] = vrows[ei, sl] * g16[hh]
                return _

            lax.fori_loop(0, CH, edge_mul, None)
            pltpu.sync_copy(vrows, wv_sh.at[didx], add=True)
            return _

        lax.fori_loop(0, per_w, body, None)
        plsc.subcore_barrier()

        def obody(j, _):
            pltpu.sync_copy(wv_sh.at[pl.ds(r0 + j * CH, CH)], vrows)
            pltpu.sync_copy(vrows, wv_out.at[pl.ds(cid * NPAD + r0 + j * CH, CH)])
            return _

        lax.fori_loop(0, nz, obody, None)

    return scatter_wv


def _make_zsum(NPAD, E, CH, per_w, NC, NS):
    f32 = jnp.float32
    mesh = plsc.VectorSubcoreMesh(core_axis_name="c", subcore_axis_name="s")
    NW = NC * NS
    ZLEN = NPAD * H

    @functools.partial(
        pl.kernel,
        out_type=jax.ShapeDtypeStruct((NW * ZLEN,), f32),
        mesh=mesh,
        scratch_types=[
            pltpu.VMEM((CH,), jnp.int32), pltpu.VMEM((CH * D,), f32),
            pltpu.VMEM((ZLEN,), f32),
        ],
        compiler_params=pltpu.CompilerParams(needs_layout_passes=False),
    )
    def zsum(dst2d, gatesf_hbm, zzf_hbm, z_out, didx, grows, zpart):
        cid = lax.axis_index("c")
        sid = lax.axis_index("s")
        wid = sid * NC + cid
        lane = lax.iota(jnp.int32, 16)
        pltpu.sync_copy(zzf_hbm, zpart)

        def body(i, _):
            c = wid * per_w + i
            pltpu.sync_copy(dst2d.at[c], didx)
            pltpu.sync_copy(gatesf_hbm.at[pl.ds(c * CH * D, CH * D)], grows)

            def zqbody(q, _):
                dvec = didx[pl.ds(q * 16, 16)]
                for r in range(16):
                    dsel = jnp.take_along_axis(
                        dvec, jnp.broadcast_to(jnp.int32(r), (16,)), axis=0)
                    zidx = dsel * H + lane
                    gvals = grows[pl.ds((q * 16 + r) * D, 16)]
                    cur = plsc.load_gather(zpart, [zidx])
                    plsc.store_scatter(zpart, [zidx], cur + gvals)
                return _

            lax.fori_loop(0, CH // 16, zqbody, None)
            return _

        lax.fori_loop(0, per_w, body, None)
        pltpu.sync_copy(zpart, z_out.at[pl.ds(wid * ZLEN, ZLEN)])

    return zsum


def _combine_body(wv0, wv1, z3, ng, expm, hout_ref):
    wv = (wv0[...] + wv1[...]) * ng[...]
    z8 = jnp.sum(z3[...], axis=0)
    z = jnp.dot(z8, expm[...], precision=_HIGH)
    hout_ref[...] = wv / (z + 1e-30)


def _combine_stage(wv0, wv1, z3, ngate, expm):
    N = wv0.shape[0]
    NW = z3.shape[0]
    BN = 1024
    grid = (N // BN,)
    full = lambda r, c: pl.BlockSpec((r, c), lambda i: (0, 0))
    blk = lambda c: pl.BlockSpec((BN, c), lambda i: (i, 0))
    return pl.pallas_call(
        _combine_body,
        grid=grid,
        in_specs=[blk(OH), blk(OH),
                  pl.BlockSpec((NW, BN, H), lambda i: (0, i, 0)),
                  blk(OH), full(H, OH)],
        out_specs=blk(OH),
        out_shape=jax.ShapeDtypeStruct((N, OH), jnp.float32),
    )(wv0, wv1, z3, ngate, expm)


def kernel(h, e, edge_index, W_Q, b_Q, W_K, b_K, W_V, b_V, W_pe, b_pe,
           W_ng1, b_ng1, ln_ng_w, ln_ng_b, W_ng2, b_ng2, W_cc, b_cc,
           ln_cc_w, ln_cc_b, conv1_w, conv1_b, conv2_w, conv2_b):
    f32 = jnp.float32
    N = h.shape[0]
    E = e.shape[0]

    I8 = jnp.eye(H, dtype=f32)
    bd1 = jnp.kron(I8, W_cc[0:16, :])
    bd2 = jnp.kron(I8, W_cc[16:32, :])
    bd3 = jnp.kron(I8, W_cc[32:48, :])
    gm = jnp.kron(I8, jnp.full((D, D), 1.0 / D, dtype=f32))
    bcc = jnp.tile(b_cc, H)[None]
    lnw = jnp.tile(ln_cc_w, H)[None]
    lnb = jnp.tile(ln_cc_b, H)[None]
    eyes = [jnp.eye(D, k=1 - k, dtype=f32) for k in range(3)]
    M1 = sum(jnp.kron(conv1_w[:, :, k].T, eyes[k]) for k in range(3))
    M2 = sum(jnp.kron(conv2_w[:, :, k].T, eyes[k]) for k in range(3))
    b1v = jnp.repeat(conv1_b, D)
    b2v = jnp.repeat(conv2_b, D)
    m12 = M1 @ M2
    b12 = (b1v @ M2 + b2v)[None]
    summ = jnp.concatenate(
        [jnp.kron(I8, jnp.ones((D, 1), f32)), jnp.zeros((OH, H), f32)], axis=1)
    expm = jnp.kron(I8, jnp.ones((1, D), f32))

    info = plsc.get_sparse_core_info()
    NC, NS = info.num_cores, info.num_subcores
    NW = NC * NS
    CH = 80
    nchunk = E // CH
    per_w = nchunk // NW
    src2d = edge_index[0].reshape(nchunk, CH)
    dst2d = edge_index[1].reshape(nchunk, CH)

    qh, kh, vh, ngate = _node_stage(h, W_Q, b_Q, W_K, b_K, W_V, b_V,
                                    W_ng1, b_ng1, ln_ng_w, ln_ng_b, W_ng2, b_ng2)
    ksrc, qdst = _make_gather_kq(N, E, CH, per_w, NC, NS)(kh, qh, src2d, dst2d)
    bf16 = jnp.bfloat16
    e_out, gates = _edge_stage(e, ksrc, qdst, W_pe, b_pe,
                               bd1.astype(bf16), bd2.astype(bf16),
                               bd3.astype(bf16), bcc, gm.astype(bf16),
                               lnw, lnb, m12.astype(bf16), b12,
                               summ.astype(bf16))
    NPAD = ((N + 8 * NW - 1) // (8 * NW)) * (8 * NW)
    zw = jnp.zeros((CH, OH), f32)
    zzf = jnp.zeros((NPAD * H,), f32)
    gatesf = gates.reshape(E * D)
    wv_p = _make_scatter(NPAD, E, CH, per_w, NC, NS)(
        vh, src2d, dst2d, gatesf, zw)
    z_p = _make_zsum(NPAD, E, CH, per_w, NC, NS)(dst2d, gatesf, zzf)
    ng_pad = jnp.concatenate([ngate, jnp.zeros((NPAD - N, OH), f32)], axis=0)
    z3 = z_p.reshape(NW, NPAD, H)
    h_out = _combine_stage(wv_p[:NPAD], wv_p[NPAD:], z3, ng_pad, expm)

    return (h_out[:N].reshape(N, H, D), e_out.reshape(E, H, D))


# speedup vs baseline: 19.2392x; 1.0364x over previous
import functools
import math

import jax
import jax.numpy as jnp
import numpy as np
from jax import lax
from jax.experimental import pallas as pl
from jax.experimental.pallas import tpu as pltpu
from jax.experimental.pallas import tpu_sc as plsc

H = 8
D = 16
OH = H * D

_HIGH = lax.Precision.HIGHEST


def _node_body(h_ref, wq, bq, wk, bk, wv, bv, wng1, bng1, lnw, lnb, wng2, bng2,
               qh_ref, kh_ref, vh_ref, ng_ref):
    hb = h_ref[...]
    qh_ref[...] = jnp.dot(hb, wq[...], precision=_HIGH) + bq[...]
    kh_ref[...] = jnp.dot(hb, wk[...], precision=_HIGH) + bk[...]
    vh_ref[...] = jnp.dot(hb, wv[...], precision=_HIGH) + bv[...]
    a = jnp.dot(hb, wng1[...], precision=_HIGH) + bng1[...]
    mu = jnp.mean(a, axis=-1, keepdims=True)
    var = jnp.mean((a - mu) ** 2, axis=-1, keepdims=True)
    a = (a - mu) * lax.rsqrt(var + 1e-5) * lnw[...] + lnb[...]
    ng = jnp.dot(jnp.tanh(a), wng2[...], precision=_HIGH) + bng2[...]
    ng_ref[...] = jax.nn.sigmoid(ng)


def _node_stage(h, W_Q, b_Q, W_K, b_K, W_V, b_V, W_ng1, b_ng1, lnw, lnb, W_ng2, b_ng2):
    N = h.shape[0]
    BN = 1000
    grid = (N // BN,)
    f32 = jnp.float32
    full = lambda r, c: pl.BlockSpec((r, c), lambda i: (0, 0))
    blk = lambda c: pl.BlockSpec((BN, c), lambda i: (i, 0))
    out_shapes = tuple(jax.ShapeDtypeStruct((N, OH), f32) for _ in range(4))
    return pl.pallas_call(
        _node_body,
        grid=grid,
        in_specs=[blk(128), full(128, OH), full(1, OH), full(128, OH), full(1, OH),
                  full(128, OH), full(1, OH), full(128, 64), full(1, 64),
                  full(1, 64), full(1, 64), full(64, OH), full(1, OH)],
        out_specs=tuple(blk(OH) for _ in range(4)),
        out_shape=out_shapes,
    )(h, W_Q, b_Q[None], W_K, b_K[None], W_V, b_V[None], W_ng1, b_ng1[None],
      lnw[None], lnb[None], W_ng2, b_ng2[None])


def _make_gather_kq(N, E, CH, per_w, NC, NS):
    f32 = jnp.float32
    mesh = plsc.VectorSubcoreMesh(core_axis_name="c", subcore_axis_name="s")

    @functools.partial(
        pl.kernel,
        out_type=(jax.ShapeDtypeStruct((E, OH), f32),
                  jax.ShapeDtypeStruct((E, OH), f32)),
        mesh=mesh,
        scratch_types=[
            pltpu.VMEM((CH,), jnp.int32), pltpu.VMEM((CH,), jnp.int32),
            pltpu.VMEM((CH, OH), f32), pltpu.VMEM((CH, OH), f32),
            pltpu.SemaphoreType.DMA, pltpu.SemaphoreType.DMA,
        ],
    )
    def gather_kq(kh_hbm, qh_hbm, src2d, dst2d, ksrc_hbm, qdst_hbm,
                  sidx, didx, krows, qrows, sem1, sem2):
        wid = lax.axis_index("s") * NC + lax.axis_index("c")

        def body(i, _):
            c = wid * per_w + i
            pltpu.sync_copy(src2d.at[c], sidx)
            pltpu.sync_copy(dst2d.at[c], didx)
            cp1 = pltpu.async_copy(kh_hbm.at[sidx], krows, sem1)
            cp2 = pltpu.async_copy(qh_hbm.at[didx], qrows, sem2)
            cp1.wait()
            cp2.wait()
            pltpu.sync_copy(krows, ksrc_hbm.at[pl.ds(c * CH, CH)])
            pltpu.sync_copy(qrows, qdst_hbm.at[pl.ds(c * CH, CH)])
            return _

        lax.fori_loop(0, per_w, body, None)

    return gather_kq


def _edge_body(e_ref, k_ref, q_ref, wpe, bpe, bd1, bd2, bd3, bcc, gm, lnw, lnb,
               m12, b12, summ, eout_ref, gate_ref):
    bf16 = jnp.bfloat16
    f32 = jnp.float32
    K = k_ref[...]
    Q = q_ref[...]
    P = jnp.dot(e_ref[...], wpe[...]) + bpe[...]
    score = K * Q * P * 0.25
    eout_ref[...] = score
    cc = (jnp.dot(K.astype(bf16), bd1[...], preferred_element_type=f32)
          + jnp.dot(Q.astype(bf16), bd2[...], preferred_element_type=f32)
          + jnp.dot(P.astype(bf16), bd3[...], preferred_element_type=f32)
          + bcc[...])
    g = gm[...]
    mu = jnp.dot(cc.astype(bf16), g, preferred_element_type=f32)
    var = jnp.dot((cc * cc).astype(bf16), g, preferred_element_type=f32) - mu * mu
    t = jnp.tanh((cc - mu) * lax.rsqrt(var + 1e-5) * lnw[...] + lnb[...])
    u = jnp.dot(t.astype(bf16), m12[...], preferred_element_type=f32) + b12[...]
    gs = score * jax.nn.sigmoid(u)
    s = jnp.dot(gs.astype(bf16), summ[...], preferred_element_type=f32)
    s = jnp.clip(s, -0.005, 0.005)
    col = lax.broadcasted_iota(jnp.int32, s.shape, 1)
    gate_ref[...] = jnp.where(col < H, jnp.exp(s), 0.0)


def _edge_stage(e, ksrc, qdst, W_pe, b_pe, bd1, bd2, bd3, bcc, gm, lnw, lnb,
                m12, b12, summ):
    E = e.shape[0]
    BE = 512
    grid = (E // BE,)
    f32 = jnp.float32
    full = lambda r, c: pl.BlockSpec((r, c), lambda i: (0, 0))
    blk = lambda c: pl.BlockSpec((BE, c), lambda i: (i, 0))
    return pl.pallas_call(
        _edge_body,
        grid=grid,
        in_specs=[blk(128), blk(OH), blk(OH), full(128, OH), full(1, OH),
                  full(OH, OH), full(OH, OH), full(OH, OH), full(1, OH),
                  full(OH, OH), full(1, OH), full(1, OH), full(OH, OH),
                  full(1, OH), full(OH, D)],
        out_specs=(blk(OH), blk(D)),
        out_shape=(jax.ShapeDtypeStruct((E, OH), f32),
                   jax.ShapeDtypeStruct((E, D), f32)),
    )(e, ksrc, qdst, W_pe, b_pe[None], bd1, bd2, bd3, bcc, gm, lnw, lnb, m12,
      b12, summ)


def _make_scatter(NPAD, E, CH, per_w, NC, NS):
    f32 = jnp.float32
    mesh = plsc.VectorSubcoreMesh(core_axis_name="c", subcore_axis_name="s")
    rows_per_tile = NPAD // NS

    @functools.partial(
        pl.kernel,
        out_type=jax.ShapeDtypeStruct((NC * NPAD, OH), f32),
        mesh=mesh,
        scratch_types=[
            pltpu.VMEM((CH,), jnp.int32), pltpu.VMEM((CH,), jnp.int32),
            pltpu.VMEM((CH, OH), f32), pltpu.VMEM((CH * D,), f32),
            pltpu.VMEM_SHARED((NPAD, OH), f32),
            pltpu.SemaphoreType.DMA,
        ],
    )
    def scatter_wv(vh_hbm, src2d, dst2d, gatesf_hbm, zw_hbm,
                   wv_out, sidx, didx, vrows, grows, wv_sh, sem):
        cid = lax.axis_index("c")
        sid = lax.axis_index("s")
        wid = sid * NC + cid
        r0 = sid * rows_per_tile
        nz = rows_per_tile // CH
        pltpu.sync_copy(zw_hbm, vrows)

        def zbody(j, _):
            pltpu.sync_copy(vrows, wv_sh.at[pl.ds(r0 + j * CH, CH)])
            return _

        lax.fori_loop(0, nz, zbody, None)
        plsc.subcore_barrier()

        def body(i, _):
            c = wid * per_w + i
            pltpu.sync_copy(src2d.at[c], sidx)
            pltpu.sync_copy(dst2d.at[c], didx)
            cp = pltpu.async_copy(vh_hbm.at[sidx], vrows, sem)
            pltpu.sync_copy(gatesf_hbm.at[pl.ds(c * CH * D, CH * D)], grows)
            cp.wait()

            def edge_mul(ei, _):
                g16 = grows[pl.ds(ei * D, D)]
                for hh in range(H):
                    sl = pl.ds(hh * D, D)
                    vrows[ei, sl] = vrows[ei, sl] * g16[hh]
                return _

            lax.fori_loop(0, CH, edge_mul, None)
            pltpu.sync_copy(vrows, wv_sh.at[didx], add=True)
            return _

        lax.fori_loop(0, per_w, body, None)
        plsc.subcore_barrier()

        def obody(j, _):
            pltpu.sync_copy(wv_sh.at[pl.ds(r0 + j * CH, CH)], vrows)
            pltpu.sync_copy(vrows, wv_out.at[pl.ds(cid * NPAD + r0 + j * CH, CH)])
            return _

        lax.fori_loop(0, nz, obody, None)

    return scatter_wv


def _make_zsum(NPAD, E, CH, per_w, NC, NS):
    f32 = jnp.float32
    mesh = plsc.VectorSubcoreMesh(core_axis_name="c", subcore_axis_name="s")
    NW = NC * NS
    ZLEN = NPAD * H

    @functools.partial(
        pl.kernel,
        out_type=jax.ShapeDtypeStruct((NW * ZLEN,), f32),
        mesh=mesh,
        scratch_types=[
            pltpu.VMEM((CH,), jnp.int32), pltpu.VMEM((CH * D,), f32),
            pltpu.VMEM((ZLEN,), f32),
        ],
        compiler_params=pltpu.CompilerParams(needs_layout_passes=False),
    )
    def zsum(dst2d, gatesf_hbm, zzf_hbm, z_out, didx, grows, zpart):
        cid = lax.axis_index("c")
        sid = lax.axis_index("s")
        wid = sid * NC + cid
        lane = lax.iota(jnp.int32, 16)
        pltpu.sync_copy(zzf_hbm, zpart)

        def body(i, _):
            c = wid * per_w + i
            pltpu.sync_copy(dst2d.at[c], didx)
            pltpu.sync_copy(gatesf_hbm.at[pl.ds(c * CH * D, CH * D)], grows)

            def zqbody(q, _):
                dvec = didx[pl.ds(q * 16, 16)]
                for r in range(16):
                    dsel = jnp.take_along_axis(
                        dvec, jnp.broadcast_to(jnp.int32(r), (16,)), axis=0)
                    zidx = dsel * H + lane
                    gvals = grows[pl.ds((q * 16 + r) * D, 16)]
                    cur = plsc.load_gather(zpart, [zidx])
                    plsc.store_scatter(zpart, [zidx], cur + gvals)
                return _

            lax.fori_loop(0, CH // 16, zqbody, None)
            return _

        lax.fori_loop(0, per_w, body, None)
        pltpu.sync_copy(zpart, z_out.at[pl.ds(wid * ZLEN, ZLEN)])

    return zsum


def _combine_body(wv0, wv1, z3, ng, expm, hout_ref):
    wv = (wv0[...] + wv1[...]) * ng[...]
    z8 = jnp.sum(z3[...], axis=0)
    z = jnp.dot(z8, expm[...], precision=_HIGH)
    hout_ref[...] = wv / (z + 1e-30)


def _combine_stage(wv0, wv1, z3, ngate, expm):
    N = wv0.shape[0]
    NW = z3.shape[0]
    BN = 1024
    grid = (N // BN,)
    full = lambda r, c: pl.BlockSpec((r, c), lambda i: (0, 0))
    blk = lambda c: pl.BlockSpec((BN, c), lambda i: (i, 0))
    return pl.pallas_call(
        _combine_body,
        grid=grid,
        in_specs=[blk(OH), blk(OH),
                  pl.BlockSpec((NW, BN, H), lambda i: (0, i, 0)),
                  blk(OH), full(H, OH)],
        out_specs=blk(OH),
        out_shape=jax.ShapeDtypeStruct((N, OH), jnp.float32),
    )(wv0, wv1, z3, ngate, expm)


def kernel(h, e, edge_index, W_Q, b_Q, W_K, b_K, W_V, b_V, W_pe, b_pe,
           W_ng1, b_ng1, ln_ng_w, ln_ng_b, W_ng2, b_ng2, W_cc, b_cc,
           ln_cc_w, ln_cc_b, conv1_w, conv1_b, conv2_w, conv2_b):
    f32 = jnp.float32
    N = h.shape[0]
    E = e.shape[0]

    I8 = jnp.eye(H, dtype=f32)
    bd1 = jnp.kron(I8, W_cc[0:16, :])
    bd2 = jnp.kron(I8, W_cc[16:32, :])
    bd3 = jnp.kron(I8, W_cc[32:48, :])
    gm = jnp.kron(I8, jnp.full((D, D), 1.0 / D, dtype=f32))
    bcc = jnp.tile(b_cc, H)[None]
    lnw = jnp.tile(ln_cc_w, H)[None]
    lnb = jnp.tile(ln_cc_b, H)[None]
    eyes = [jnp.eye(D, k=1 - k, dtype=f32) for k in range(3)]
    M1 = sum(jnp.kron(conv1_w[:, :, k].T, eyes[k]) for k in range(3))
    M2 = sum(jnp.kron(conv2_w[:, :, k].T, eyes[k]) for k in range(3))
    b1v = jnp.repeat(conv1_b, D)
    b2v = jnp.repeat(conv2_b, D)
    m12 = M1 @ M2
    b12 = (b1v @ M2 + b2v)[None]
    summ = jnp.concatenate(
        [jnp.kron(I8, jnp.ones((D, 1), f32)), jnp.zeros((OH, H), f32)], axis=1)
    expm = jnp.kron(I8, jnp.ones((1, D), f32))

    info = plsc.get_sparse_core_info()
    NC, NS = info.num_cores, info.num_subcores
    NW = NC * NS
    CH = 80
    nchunk = E // CH
    per_w = nchunk // NW
    src2d = edge_index[0].reshape(nchunk, CH)
    dst2d = edge_index[1].reshape(nchunk, CH)

    qh, kh, vh, ngate = _node_stage(h, W_Q, b_Q, W_K, b_K, W_V, b_V,
                                    W_ng1, b_ng1, ln_ng_w, ln_ng_b, W_ng2, b_ng2)
    ksrc, qdst = _make_gather_kq(N, E, CH, per_w, NC, NS)(kh, qh, src2d, dst2d)
    bf16 = jnp.bfloat16
    e_out, gates = _edge_stage(e, ksrc, qdst, W_pe, b_pe,
                               bd1.astype(bf16), bd2.astype(bf16),
                               bd3.astype(bf16), bcc, gm.astype(bf16),
                               lnw, lnb, m12.astype(bf16), b12,
                               summ.astype(bf16))
    NPAD = ((N + 8 * NW - 1) // (8 * NW)) * (8 * NW)
    zw = jnp.zeros((CH, OH), f32)
    zzf = jnp.zeros((NPAD * H,), f32)
    gatesf = gates.reshape(E * D)
    wv_p = _make_scatter(NPAD, E, CH, per_w, NC, NS)(
        vh, src2d, dst2d, gatesf, zw)
    z_p = _make_zsum(NPAD, E, CH, per_w, NC, NS)(dst2d, gatesf, zzf)
    ng_pad = jnp.concatenate([ngate, jnp.zeros((NPAD - N, OH), f32)], axis=0)
    z3 = z_p.reshape(NW, NPAD, H)
    h_out = _combine_stage(wv_p[:NPAD], wv_p[NPAD:], z3, ng_pad, expm)

    return (h_out[:N].reshape(N, H, D), e_out.reshape(E, H, D))
